# Initial kernel scaffold; baseline (speedup 1.0000x reference)
import jax
import jax.numpy as jnp
from jax.experimental import pallas as pl


def kernel(x, edge_index, batch, W1, b1, W2, b2, W3, b3, Wf, bf, Wp, bp):
    raise NotImplementedError("write your pallas kernel here")


# speedup vs baseline: 16.8849x; 16.8849x over previous
import functools

import jax
import jax.numpy as jnp
from jax import lax
from jax.experimental import pallas as pl
from jax.experimental.pallas import tpu as pltpu
from jax.experimental.pallas import tpu_sc as plsc

N_NODES = 10000
N_EDGES = 320000
D_IN = 128
HID = 64
N_GRAPHS = 100

NC = 2
NS = 16
NW = NC * NS
EDGES_PER_W = N_EDGES // NW
CHUNK = 125
CHUNKS_PER_W = EDGES_PER_W // CHUNK
ROWS_PER_TILE = N_NODES // NS
K = 4
SUPER = CHUNKS_PER_W // (2 * K)


def _sc_agg_body(p_hbm, src_hbm, dst_hbm, zeros_hbm, out_hbm,
                 acc_sh, src_v, dst_v, rows_v, gsem0, gsem1, ssem0, ssem1):
    c = lax.axis_index("c")
    s = lax.axis_index("s")
    wid = c * NS + s
    rstart = s * ROWS_PER_TILE
    pltpu.sync_copy(zeros_hbm.at[pl.ds(rstart, ROWS_PER_TILE)],
                    acc_sh.at[pl.ds(rstart, ROWS_PER_TILE)])
    pltpu.sync_copy(src_hbm.at[pl.ds(wid * CHUNKS_PER_W, CHUNKS_PER_W)], src_v)
    pltpu.sync_copy(dst_hbm.at[pl.ds(wid * CHUNKS_PER_W, CHUNKS_PER_W)], dst_v)
    plsc.subcore_barrier()

    def drain(n, sem):
        for _ in range(n):
            pltpu.make_async_copy(p_hbm.at[pl.ds(0, CHUNK)], rows_v.at[0],
                                  sem).wait()

    def fire_gathers(j0, bank, sem):
        for i in range(K):
            pltpu.async_copy(p_hbm.at[src_v.at[j0 + i]],
                             rows_v.at[bank * K + i], sem)

    def fire_scatters(j0, bank, sem):
        for i in range(K):
            pltpu.async_copy(rows_v.at[bank * K + i],
                             acc_sh.at[dst_v.at[j0 + i]], sem, add=True)

    def body(g, carry):
        j0 = g * 2 * K

        @pl.when(g > 0)
        def _():
            drain(K, ssem1)
        fire_gathers(j0 + K, 1, gsem1)
        drain(K, gsem0)
        fire_scatters(j0, 0, ssem0)
        drain(K, ssem0)

        @pl.when(g < SUPER - 1)
        def _():
            fire_gathers(j0 + 2 * K, 0, gsem0)
        drain(K, gsem1)
        fire_scatters(j0 + K, 1, ssem1)
        return carry

    fire_gathers(0, 0, gsem0)
    lax.fori_loop(0, SUPER, body, 0, unroll=False)
    drain(K, ssem1)
    plsc.subcore_barrier()
    pltpu.sync_copy(acc_sh.at[pl.ds(rstart, ROWS_PER_TILE)],
                    out_hbm.at[c].at[pl.ds(rstart, ROWS_PER_TILE)])


@functools.cache
def _sc_agg():
    return pl.kernel(
        _sc_agg_body,
        out_type=jax.ShapeDtypeStruct((NC, N_NODES, HID), jnp.float32),
        mesh=plsc.VectorSubcoreMesh(core_axis_name="c", subcore_axis_name="s",
                                    num_cores=NC, num_subcores=NS),
        scratch_types=[
            pltpu.VMEM_SHARED((N_NODES, HID), jnp.float32),
            pltpu.VMEM((CHUNKS_PER_W, CHUNK), jnp.int32),
            pltpu.VMEM((CHUNKS_PER_W, CHUNK), jnp.int32),
            pltpu.VMEM((2 * K, CHUNK, HID), jnp.float32),
            pltpu.SemaphoreType.DMA,
            pltpu.SemaphoreType.DMA,
            pltpu.SemaphoreType.DMA,
            pltpu.SemaphoreType.DMA,
        ],
        compiler_params=pltpu.CompilerParams(use_tc_tiling_on_sc=False),
    )


def _mm_body(x_ref, w_ref, o_ref):
    o_ref[...] = jnp.dot(x_ref[...], w_ref[...],
                         preferred_element_type=jnp.float32)


def _combine_body(p_ref, parts_ref, b_ref, w_ref, o_ref):
    h = p_ref[...] + parts_ref[0] + parts_ref[1] + b_ref[...]
    h = jnp.maximum(h, 0.0)
    o_ref[...] = jnp.dot(h, w_ref[...], preferred_element_type=jnp.float32)


def _final_body(p_ref, parts_ref, b_ref, batch_ref,
                wf_ref, bf_ref, wp_ref, bp_ref, o_ref):
    h = p_ref[...] + parts_ref[0] + parts_ref[1] + b_ref[...]
    gids = lax.broadcasted_iota(jnp.int32, (N_GRAPHS, N_NODES), 0)
    onehot = (gids == batch_ref[...]).astype(jnp.float32)
    sums = jnp.dot(onehot, h, preferred_element_type=jnp.float32)
    counts = jnp.sum(onehot, axis=1, keepdims=True)
    mean = sums / jnp.maximum(counts, 1.0)
    g = jnp.dot(mean, wf_ref[...], preferred_element_type=jnp.float32)
    g = g + bf_ref[...]
    g = jnp.dot(g, wp_ref[...], preferred_element_type=jnp.float32)
    g = g + bp_ref[...]
    o_ref[...] = jax.nn.sigmoid(g)


def _tc(body, out_shape):
    return pl.pallas_call(body, out_shape=out_shape)


_BLK = 1000


def _combine_call(p, a, b, W):
    f32 = jnp.float32
    grid = (N_NODES // _BLK,)
    return pl.pallas_call(
        _combine_body,
        grid=grid,
        in_specs=[
            pl.BlockSpec((_BLK, HID), lambda i: (i, 0)),
            pl.BlockSpec((NC, _BLK, HID), lambda i: (0, i, 0)),
            pl.BlockSpec((1, HID), lambda i: (0, 0)),
            pl.BlockSpec((HID, HID), lambda i: (0, 0)),
        ],
        out_specs=pl.BlockSpec((_BLK, HID), lambda i: (i, 0)),
        out_shape=jax.ShapeDtypeStruct((N_NODES, HID), f32),
    )(p, a, b, W)


def kernel(x, edge_index, batch, W1, b1, W2, b2, W3, b3, Wf, bf, Wp, bp):
    f32 = jnp.float32
    src = edge_index[0].astype(jnp.int32).reshape(N_EDGES // CHUNK, CHUNK)
    dst = edge_index[1].astype(jnp.int32).reshape(N_EDGES // CHUNK, CHUNK)
    zeros = jnp.zeros((N_NODES, HID), f32)
    batch2d = batch.astype(jnp.int32).reshape(1, N_NODES)
    b1r, b2r, b3r = b1.reshape(1, HID), b2.reshape(1, HID), b3.reshape(1, HID)
    bfr, bpr = bf.reshape(1, 32), bp.reshape(1, 1)

    p1 = _tc(_mm_body, jax.ShapeDtypeStruct((N_NODES, HID), f32))(x, W1)
    a1 = _sc_agg()(p1, src, dst, zeros)
    p2 = _combine_call(p1, a1, b1r, W2)
    a2 = _sc_agg()(p2, src, dst, zeros)
    p3 = _combine_call(p2, a2, b2r, W3)
    a3 = _sc_agg()(p3, src, dst, zeros)
    out = _tc(_final_body, jax.ShapeDtypeStruct((N_GRAPHS, 1), f32))(
        p3, a3, b3r, batch2d, Wf, bfr, Wp, bpr)
    return out


# speedup vs baseline: 17.1564x; 1.0161x over previous
import functools

import jax
import jax.numpy as jnp
from jax import lax
from jax.experimental import pallas as pl
from jax.experimental.pallas import tpu as pltpu
from jax.experimental.pallas import tpu_sc as plsc

N_NODES = 10000
N_EDGES = 320000
D_IN = 128
HID = 64
N_GRAPHS = 100

NC = 2
NS = 16
NW = NC * NS
EDGES_PER_W = N_EDGES // NW
CHUNK = 80
CHUNKS_PER_W = EDGES_PER_W // CHUNK
ROWS_PER_TILE = N_NODES // NS
K = 2
SUPER = (CHUNKS_PER_W - 1) // (2 * K)
TAIL = SUPER * 2 * K


def _sc_agg_body(p_hbm, ei_hbm, zeros_hbm, out_hbm,
                 acc_sh, src_v, dst_v, rows_v, gsem0, gsem1, ssem0, ssem1):
    c = lax.axis_index("c")
    s = lax.axis_index("s")
    wid = c * NS + s
    rstart = s * ROWS_PER_TILE
    pltpu.sync_copy(zeros_hbm.at[pl.ds(rstart, ROWS_PER_TILE)],
                    acc_sh.at[pl.ds(rstart, ROWS_PER_TILE)])
    pltpu.sync_copy(ei_hbm.at[0].at[pl.ds(wid * EDGES_PER_W, EDGES_PER_W)],
                    src_v)
    pltpu.sync_copy(ei_hbm.at[1].at[pl.ds(wid * EDGES_PER_W, EDGES_PER_W)],
                    dst_v)
    plsc.subcore_barrier()

    def idx(j):
        return pl.ds(pl.multiple_of(j * CHUNK, 8), CHUNK)

    def drain(n, sem):
        for _ in range(n):
            pltpu.make_async_copy(p_hbm.at[pl.ds(0, CHUNK)], rows_v.at[0],
                                  sem).wait()

    def fire_gathers(j0, bank, sem):
        for i in range(K):
            pltpu.async_copy(p_hbm.at[src_v.at[idx(j0 + i)]],
                             rows_v.at[bank * K + i], sem)

    def fire_scatters(j0, bank, sem):
        for i in range(K):
            pltpu.async_copy(rows_v.at[bank * K + i],
                             acc_sh.at[dst_v.at[idx(j0 + i)]], sem, add=True)

    def body(g, carry):
        j0 = g * 2 * K

        @pl.when(g > 0)
        def _():
            drain(K, ssem1)
        fire_gathers(j0 + K, 1, gsem1)
        drain(K, gsem0)
        fire_scatters(j0, 0, ssem0)
        drain(K, ssem0)

        @pl.when(g < SUPER - 1)
        def _():
            fire_gathers(j0 + 2 * K, 0, gsem0)
        drain(K, gsem1)
        fire_scatters(j0 + K, 1, ssem1)
        return carry

    fire_gathers(0, 0, gsem0)
    lax.fori_loop(0, SUPER, body, 0, unroll=False)
    drain(K, ssem1)
    pltpu.async_copy(p_hbm.at[src_v.at[idx(TAIL)]], rows_v.at[0], gsem0)
    drain(1, gsem0)
    pltpu.async_copy(rows_v.at[0], acc_sh.at[dst_v.at[idx(TAIL)]],
                     ssem0, add=True)
    drain(1, ssem0)
    plsc.subcore_barrier()
    pltpu.sync_copy(acc_sh.at[pl.ds(rstart, ROWS_PER_TILE)],
                    out_hbm.at[c].at[pl.ds(rstart, ROWS_PER_TILE)])


@functools.cache
def _sc_agg():
    return pl.kernel(
        _sc_agg_body,
        out_type=jax.ShapeDtypeStruct((NC, N_NODES, HID), jnp.float32),
        mesh=plsc.VectorSubcoreMesh(core_axis_name="c", subcore_axis_name="s",
                                    num_cores=NC, num_subcores=NS),
        scratch_types=[
            pltpu.VMEM_SHARED((N_NODES, HID), jnp.float32),
            pltpu.VMEM((EDGES_PER_W,), jnp.int32),
            pltpu.VMEM((EDGES_PER_W,), jnp.int32),
            pltpu.VMEM((2 * K, CHUNK, HID), jnp.float32),
            pltpu.SemaphoreType.DMA,
            pltpu.SemaphoreType.DMA,
            pltpu.SemaphoreType.DMA,
            pltpu.SemaphoreType.DMA,
        ],
        compiler_params=pltpu.CompilerParams(use_tc_tiling_on_sc=False),
    )


def _mm_body(x_ref, w_ref, o_ref):
    o_ref[...] = jnp.dot(x_ref[...], w_ref[...],
                         preferred_element_type=jnp.float32)


def _combine_body(p_ref, parts_ref, b_ref, w_ref, o_ref):
    h = p_ref[...] + parts_ref[0] + parts_ref[1] + b_ref[...]
    h = jnp.maximum(h, 0.0)
    o_ref[...] = jnp.dot(h, w_ref[...], preferred_element_type=jnp.float32)


def _final_body(p_ref, parts_ref, b_ref, batch_ref,
                wf_ref, bf_ref, wp_ref, bp_ref, o_ref):
    h = p_ref[...] + parts_ref[0] + parts_ref[1] + b_ref[...]
    gids = lax.broadcasted_iota(jnp.int32, (N_GRAPHS, N_NODES), 0)
    onehot = (gids == batch_ref[...]).astype(jnp.float32)
    sums = jnp.dot(onehot, h, preferred_element_type=jnp.float32)
    counts = jnp.sum(onehot, axis=1, keepdims=True)
    mean = sums / jnp.maximum(counts, 1.0)
    g = jnp.dot(mean, wf_ref[...], preferred_element_type=jnp.float32)
    g = g + bf_ref[...]
    g = jnp.dot(g, wp_ref[...], preferred_element_type=jnp.float32)
    g = g + bp_ref[...]
    o_ref[...] = jax.nn.sigmoid(g)


def _tc(body, out_shape):
    return pl.pallas_call(body, out_shape=out_shape)


def kernel(x, edge_index, batch, W1, b1, W2, b2, W3, b3, Wf, bf, Wp, bp):
    f32 = jnp.float32
    ei = edge_index.astype(jnp.int32)
    zeros = jnp.zeros((N_NODES, HID), f32)
    batch2d = batch.astype(jnp.int32).reshape(1, N_NODES)
    b1r, b2r, b3r = b1.reshape(1, HID), b2.reshape(1, HID), b3.reshape(1, HID)
    bfr, bpr = bf.reshape(1, 32), bp.reshape(1, 1)

    p1 = _tc(_mm_body, jax.ShapeDtypeStruct((N_NODES, HID), f32))(x, W1)
    a1 = _sc_agg()(p1, ei, zeros)
    p2 = _tc(_combine_body,
             jax.ShapeDtypeStruct((N_NODES, HID), f32))(p1, a1, b1r, W2)
    a2 = _sc_agg()(p2, ei, zeros)
    p3 = _tc(_combine_body,
             jax.ShapeDtypeStruct((N_NODES, HID), f32))(p2, a2, b2r, W3)
    a3 = _sc_agg()(p3, ei, zeros)
    out = _tc(_final_body, jax.ShapeDtypeStruct((N_GRAPHS, 1), f32))(
        p3, a3, b3r, batch2d, Wf, bfr, Wp, bpr)
    return out


# speedup vs baseline: 17.6112x; 1.0265x over previous
import functools

import jax
import jax.numpy as jnp
from jax import lax
from jax.experimental import pallas as pl
from jax.experimental.pallas import tpu as pltpu
from jax.experimental.pallas import tpu_sc as plsc

N_NODES = 10000
N_EDGES = 320000
D_IN = 128
HID = 64
N_GRAPHS = 100

NC = 2
NS = 16
NW = NC * NS
EDGES_PER_W = N_EDGES // NW
CHUNK = 80
CHUNKS_PER_W = EDGES_PER_W // CHUNK
ROWS_PER_TILE = N_NODES // NS
K = 5
SUPER = CHUNKS_PER_W // (2 * K)
REM = CHUNKS_PER_W - SUPER * 2 * K


def _sc_agg_body(p_hbm, ei_hbm, zeros_hbm, out_hbm,
                 acc_sh, src_v, dst_v, rows_v, gsem0, gsem1, ssem0, ssem1):
    c = lax.axis_index("c")
    s = lax.axis_index("s")
    wid = c * NS + s
    rstart = s * ROWS_PER_TILE
    pltpu.sync_copy(zeros_hbm.at[pl.ds(rstart, ROWS_PER_TILE)],
                    acc_sh.at[pl.ds(rstart, ROWS_PER_TILE)])
    pltpu.sync_copy(ei_hbm.at[0].at[pl.ds(wid * EDGES_PER_W, EDGES_PER_W)],
                    src_v)
    pltpu.sync_copy(ei_hbm.at[1].at[pl.ds(wid * EDGES_PER_W, EDGES_PER_W)],
                    dst_v)
    plsc.subcore_barrier()

    def idx(j):
        return pl.ds(pl.multiple_of(j * CHUNK, 8), CHUNK)

    def drain(n, sem):
        for _ in range(n):
            pltpu.make_async_copy(p_hbm.at[pl.ds(0, CHUNK)], rows_v.at[0],
                                  sem).wait()

    def fire_gathers(j0, bank, sem):
        for i in range(K):
            pltpu.async_copy(p_hbm.at[src_v.at[idx(j0 + i)]],
                             rows_v.at[bank * K + i], sem)

    def fire_scatters(j0, bank, sem):
        for i in range(K):
            pltpu.async_copy(rows_v.at[bank * K + i],
                             acc_sh.at[dst_v.at[idx(j0 + i)]], sem, add=True)

    def body(g, carry):
        j0 = g * 2 * K

        @pl.when(g > 0)
        def _():
            drain(K, ssem1)
        fire_gathers(j0 + K, 1, gsem1)
        drain(K, gsem0)
        fire_scatters(j0, 0, ssem0)
        drain(K, ssem0)

        @pl.when(g < SUPER - 1)
        def _():
            fire_gathers(j0 + 2 * K, 0, gsem0)
        drain(K, gsem1)
        fire_scatters(j0 + K, 1, ssem1)
        return carry

    fire_gathers(0, 0, gsem0)
    lax.fori_loop(0, SUPER, body, 0, unroll=False)
    drain(K, ssem1)
    for r in range(REM):
        pltpu.async_copy(p_hbm.at[src_v.at[idx(SUPER * 2 * K + r)]],
                         rows_v.at[0], gsem0)
        drain(1, gsem0)
        pltpu.async_copy(rows_v.at[0],
                         acc_sh.at[dst_v.at[idx(SUPER * 2 * K + r)]],
                         ssem0, add=True)
        drain(1, ssem0)
    plsc.subcore_barrier()
    pltpu.sync_copy(acc_sh.at[pl.ds(rstart, ROWS_PER_TILE)],
                    out_hbm.at[c].at[pl.ds(rstart, ROWS_PER_TILE)])


@functools.cache
def _sc_agg():
    return pl.kernel(
        _sc_agg_body,
        out_type=jax.ShapeDtypeStruct((NC, N_NODES, HID), jnp.float32),
        mesh=plsc.VectorSubcoreMesh(core_axis_name="c", subcore_axis_name="s",
                                    num_cores=NC, num_subcores=NS),
        scratch_types=[
            pltpu.VMEM_SHARED((N_NODES, HID), jnp.float32),
            pltpu.VMEM((EDGES_PER_W,), jnp.int32),
            pltpu.VMEM((EDGES_PER_W,), jnp.int32),
            pltpu.VMEM((2 * K, CHUNK, HID), jnp.float32),
            pltpu.SemaphoreType.DMA,
            pltpu.SemaphoreType.DMA,
            pltpu.SemaphoreType.DMA,
            pltpu.SemaphoreType.DMA,
        ],
        compiler_params=pltpu.CompilerParams(use_tc_tiling_on_sc=False),
    )


def _mm_body(x_ref, w_ref, o_ref):
    o_ref[...] = jnp.dot(x_ref[...], w_ref[...],
                         preferred_element_type=jnp.float32)


def _combine_body(p_ref, parts_ref, b_ref, w_ref, o_ref):
    h = p_ref[...] + parts_ref[0] + parts_ref[1] + b_ref[...]
    h = jnp.maximum(h, 0.0)
    o_ref[...] = jnp.dot(h, w_ref[...], preferred_element_type=jnp.float32)


def _final_body(p_ref, parts_ref, b_ref, batch_ref,
                wf_ref, bf_ref, wp_ref, bp_ref, o_ref):
    h = p_ref[...] + parts_ref[0] + parts_ref[1] + b_ref[...]
    gids = lax.broadcasted_iota(jnp.int32, (N_GRAPHS, N_NODES), 0)
    onehot = (gids == batch_ref[...]).astype(jnp.float32)
    sums = jnp.dot(onehot, h, preferred_element_type=jnp.float32)
    counts = jnp.sum(onehot, axis=1, keepdims=True)
    mean = sums / jnp.maximum(counts, 1.0)
    g = jnp.dot(mean, wf_ref[...], preferred_element_type=jnp.float32)
    g = g + bf_ref[...]
    g = jnp.dot(g, wp_ref[...], preferred_element_type=jnp.float32)
    g = g + bp_ref[...]
    o_ref[...] = jax.nn.sigmoid(g)


def _tc(body, out_shape):
    return pl.pallas_call(body, out_shape=out_shape)


def kernel(x, edge_index, batch, W1, b1, W2, b2, W3, b3, Wf, bf, Wp, bp):
    f32 = jnp.float32
    ei = edge_index.astype(jnp.int32)
    zeros = jnp.zeros((N_NODES, HID), f32)
    batch2d = batch.astype(jnp.int32).reshape(1, N_NODES)
    b1r, b2r, b3r = b1.reshape(1, HID), b2.reshape(1, HID), b3.reshape(1, HID)
    bfr, bpr = bf.reshape(1, 32), bp.reshape(1, 1)

    p1 = _tc(_mm_body, jax.ShapeDtypeStruct((N_NODES, HID), f32))(x, W1)
    a1 = _sc_agg()(p1, ei, zeros)
    p2 = _tc(_combine_body,
             jax.ShapeDtypeStruct((N_NODES, HID), f32))(p1, a1, b1r, W2)
    a2 = _sc_agg()(p2, ei, zeros)
    p3 = _tc(_combine_body,
             jax.ShapeDtypeStruct((N_NODES, HID), f32))(p2, a2, b2r, W3)
    a3 = _sc_agg()(p3, ei, zeros)
    out = _tc(_final_body, jax.ShapeDtypeStruct((N_GRAPHS, 1), f32))(
        p3, a3, b3r, batch2d, Wf, bfr, Wp, bpr)
    return out


# speedup vs baseline: 17.9136x; 1.0172x over previous
import functools

import jax
import jax.numpy as jnp
from jax import lax
from jax.experimental import pallas as pl
from jax.experimental.pallas import tpu as pltpu
from jax.experimental.pallas import tpu_sc as plsc

N_NODES = 10000
N_EDGES = 320000
D_IN = 128
HID = 64
N_GRAPHS = 100

NC = 2
NS = 16
NW = NC * NS
EDGES_PER_W = N_EDGES // NW
CHUNK = 40
CHUNKS_PER_W = EDGES_PER_W // CHUNK
ROWS_PER_TILE = N_NODES // NS
K = 5
SUPER = CHUNKS_PER_W // (2 * K)
REM = CHUNKS_PER_W - SUPER * 2 * K


def _sc_agg_body(p_hbm, ei_hbm, zeros_hbm, out_hbm,
                 acc_sh, src_v, dst_v, rows_v, gsem0, gsem1, ssem0, ssem1):
    c = lax.axis_index("c")
    s = lax.axis_index("s")
    wid = c * NS + s
    rstart = s * ROWS_PER_TILE
    pltpu.sync_copy(zeros_hbm.at[pl.ds(rstart, ROWS_PER_TILE)],
                    acc_sh.at[pl.ds(rstart, ROWS_PER_TILE)])
    pltpu.sync_copy(ei_hbm.at[0].at[pl.ds(wid * EDGES_PER_W, EDGES_PER_W)],
                    src_v)
    pltpu.sync_copy(ei_hbm.at[1].at[pl.ds(wid * EDGES_PER_W, EDGES_PER_W)],
                    dst_v)
    plsc.subcore_barrier()

    def idx(j):
        return pl.ds(pl.multiple_of(j * CHUNK, 8), CHUNK)

    def drain(n, sem):
        for _ in range(n):
            pltpu.make_async_copy(p_hbm.at[pl.ds(0, CHUNK)], rows_v.at[0],
                                  sem).wait()

    def fire_gathers(j0, bank, sem):
        for i in range(K):
            pltpu.async_copy(p_hbm.at[src_v.at[idx(j0 + i)]],
                             rows_v.at[bank * K + i], sem)

    def fire_scatters(j0, bank, sem):
        for i in range(K):
            pltpu.async_copy(rows_v.at[bank * K + i],
                             acc_sh.at[dst_v.at[idx(j0 + i)]], sem, add=True)

    def body(g, carry):
        j0 = g * 2 * K

        @pl.when(g > 0)
        def _():
            drain(K, ssem1)
        fire_gathers(j0 + K, 1, gsem1)
        drain(K, gsem0)
        fire_scatters(j0, 0, ssem0)
        drain(K, ssem0)

        @pl.when(g < SUPER - 1)
        def _():
            fire_gathers(j0 + 2 * K, 0, gsem0)
        drain(K, gsem1)
        fire_scatters(j0 + K, 1, ssem1)
        return carry

    fire_gathers(0, 0, gsem0)
    lax.fori_loop(0, SUPER, body, 0, unroll=False)
    drain(K, ssem1)
    for r in range(REM):
        pltpu.async_copy(p_hbm.at[src_v.at[idx(SUPER * 2 * K + r)]],
                         rows_v.at[0], gsem0)
        drain(1, gsem0)
        pltpu.async_copy(rows_v.at[0],
                         acc_sh.at[dst_v.at[idx(SUPER * 2 * K + r)]],
                         ssem0, add=True)
        drain(1, ssem0)
    plsc.subcore_barrier()
    pltpu.sync_copy(acc_sh.at[pl.ds(rstart, ROWS_PER_TILE)],
                    out_hbm.at[c].at[pl.ds(rstart, ROWS_PER_TILE)])


@functools.cache
def _sc_agg():
    return pl.kernel(
        _sc_agg_body,
        out_type=jax.ShapeDtypeStruct((NC, N_NODES, HID), jnp.float32),
        mesh=plsc.VectorSubcoreMesh(core_axis_name="c", subcore_axis_name="s",
                                    num_cores=NC, num_subcores=NS),
        scratch_types=[
            pltpu.VMEM_SHARED((N_NODES, HID), jnp.float32),
            pltpu.VMEM((EDGES_PER_W,), jnp.int32),
            pltpu.VMEM((EDGES_PER_W,), jnp.int32),
            pltpu.VMEM((2 * K, CHUNK, HID), jnp.float32),
            pltpu.SemaphoreType.DMA,
            pltpu.SemaphoreType.DMA,
            pltpu.SemaphoreType.DMA,
            pltpu.SemaphoreType.DMA,
        ],
        compiler_params=pltpu.CompilerParams(use_tc_tiling_on_sc=False),
    )


def _mm_body(x_ref, w_ref, o_ref):
    o_ref[...] = jnp.dot(x_ref[...], w_ref[...],
                         preferred_element_type=jnp.float32)


def _combine_body(p_ref, parts_ref, b_ref, w_ref, o_ref):
    h = p_ref[...] + parts_ref[0] + parts_ref[1] + b_ref[...]
    h = jnp.maximum(h, 0.0)
    o_ref[...] = jnp.dot(h, w_ref[...], preferred_element_type=jnp.float32)


def _final_body(p_ref, parts_ref, b_ref, batch_ref,
                wf_ref, bf_ref, wp_ref, bp_ref, o_ref):
    h = p_ref[...] + parts_ref[0] + parts_ref[1] + b_ref[...]
    gids = lax.broadcasted_iota(jnp.int32, (N_GRAPHS, N_NODES), 0)
    onehot = (gids == batch_ref[...]).astype(jnp.float32)
    sums = jnp.dot(onehot, h, preferred_element_type=jnp.float32)
    counts = jnp.sum(onehot, axis=1, keepdims=True)
    mean = sums / jnp.maximum(counts, 1.0)
    g = jnp.dot(mean, wf_ref[...], preferred_element_type=jnp.float32)
    g = g + bf_ref[...]
    g = jnp.dot(g, wp_ref[...], preferred_element_type=jnp.float32)
    g = g + bp_ref[...]
    o_ref[...] = jax.nn.sigmoid(g)


def _tc(body, out_shape):
    return pl.pallas_call(body, out_shape=out_shape)


def kernel(x, edge_index, batch, W1, b1, W2, b2, W3, b3, Wf, bf, Wp, bp):
    f32 = jnp.float32
    ei = edge_index.astype(jnp.int32)
    zeros = jnp.zeros((N_NODES, HID), f32)
    batch2d = batch.astype(jnp.int32).reshape(1, N_NODES)
    b1r, b2r, b3r = b1.reshape(1, HID), b2.reshape(1, HID), b3.reshape(1, HID)
    bfr, bpr = bf.reshape(1, 32), bp.reshape(1, 1)

    p1 = _tc(_mm_body, jax.ShapeDtypeStruct((N_NODES, HID), f32))(x, W1)
    a1 = _sc_agg()(p1, ei, zeros)
    p2 = _tc(_combine_body,
             jax.ShapeDtypeStruct((N_NODES, HID), f32))(p1, a1, b1r, W2)
    a2 = _sc_agg()(p2, ei, zeros)
    p3 = _tc(_combine_body,
             jax.ShapeDtypeStruct((N_NODES, HID), f32))(p2, a2, b2r, W3)
    a3 = _sc_agg()(p3, ei, zeros)
    out = _tc(_final_body, jax.ShapeDtypeStruct((N_GRAPHS, 1), f32))(
        p3, a3, b3r, batch2d, Wf, bfr, Wp, bpr)
    return out
